# baseline (speedup 1.0000x reference)
import jax
import jax.numpy as jnp
from jax import lax
from jax.experimental import pallas as pl
from jax.experimental.pallas import tpu as pltpu
from jax.experimental.pallas import tpu_sc as plsc

N = 10000
E = 320000
D = 128
G = 200
C = 5

NC = 2
NS = 16
NW = NC * NS
EPT = E // NW
CHUNK = 128
GCH = 26
GROUPS = 3
TAIL = EPT - GROUPS * GCH * CHUNK
RPT = 632
RPT_LAST = N - (NS - 1) * RPT


def _edge_agg_body(table_hbm, zeros_hbm, src_hbm, dst_hbm, srct_hbm,
                   dstt_hbm, out_hbm, srcv, dstv, rows0, rows1, acc,
                   gsem0, gsem1, ssem0, ssem1):
    cid = lax.axis_index("c")
    sid = lax.axis_index("s")
    wid = cid * NS + sid
    lo = sid * RPT

    @pl.when((cid == 0) & (sid < NS - 1))
    def _():
        pltpu.sync_copy(table_hbm.at[pl.ds(lo, RPT)], acc.at[pl.ds(lo, RPT)])

    @pl.when((cid == 0) & (sid == NS - 1))
    def _():
        pltpu.sync_copy(table_hbm.at[pl.ds((NS - 1) * RPT, RPT_LAST)],
                        acc.at[pl.ds((NS - 1) * RPT, RPT_LAST)])

    @pl.when((cid != 0) & (sid < NS - 1))
    def _():
        pltpu.sync_copy(zeros_hbm.at[pl.ds(lo, RPT)], acc.at[pl.ds(lo, RPT)])

    @pl.when((cid != 0) & (sid == NS - 1))
    def _():
        pltpu.sync_copy(zeros_hbm.at[pl.ds((NS - 1) * RPT, RPT_LAST)],
                        acc.at[pl.ds((NS - 1) * RPT, RPT_LAST)])

    plsc.subcore_barrier()

    @pl.loop(0, GROUPS)
    def _(g):
        pltpu.sync_copy(src_hbm.at[wid, g], srcv)
        pltpu.sync_copy(dst_hbm.at[wid, g], dstv)
        pltpu.async_copy(table_hbm.at[srcv.at[0]], rows0, gsem0)
        pltpu.async_copy(table_hbm.at[srcv.at[1]], rows1, gsem1)

        @pl.loop(0, GCH, step=2)
        def _(j):
            pltpu.make_async_copy(
                table_hbm.at[srcv.at[j]], rows0, gsem0).wait()
            pltpu.async_copy(rows0, acc.at[dstv.at[j]], ssem0, add=True)
            pltpu.make_async_copy(
                table_hbm.at[srcv.at[j + 1]], rows1, gsem1).wait()
            pltpu.async_copy(rows1, acc.at[dstv.at[j + 1]], ssem1, add=True)
            pltpu.make_async_copy(rows0, acc.at[dstv.at[j]], ssem0).wait()

            @pl.when(j + 2 < GCH)
            def _():
                pltpu.async_copy(table_hbm.at[srcv.at[j + 2]], rows0, gsem0)
            pltpu.make_async_copy(rows1, acc.at[dstv.at[j + 1]], ssem1).wait()

            @pl.when(j + 3 < GCH)
            def _():
                pltpu.async_copy(table_hbm.at[srcv.at[j + 3]], rows1, gsem1)

    pltpu.sync_copy(srct_hbm.at[wid], srcv.at[0, pl.ds(0, TAIL)])
    pltpu.sync_copy(dstt_hbm.at[wid], dstv.at[0, pl.ds(0, TAIL)])
    pltpu.sync_copy(table_hbm.at[srcv.at[0, pl.ds(0, TAIL)]],
                    rows0.at[pl.ds(0, TAIL)])
    pltpu.sync_copy(rows0.at[pl.ds(0, TAIL)],
                    acc.at[dstv.at[0, pl.ds(0, TAIL)]], add=True)

    plsc.subcore_barrier()

    @pl.when(sid < NS - 1)
    def _():
        pltpu.sync_copy(acc.at[pl.ds(lo, RPT)],
                        out_hbm.at[cid, pl.ds(lo, RPT)])

    @pl.when(sid == NS - 1)
    def _():
        pltpu.sync_copy(acc.at[pl.ds((NS - 1) * RPT, RPT_LAST)],
                        out_hbm.at[cid, pl.ds((NS - 1) * RPT, RPT_LAST)])


def _edge_agg(table, zeros, srcf, dstf, srct, dstt):
    mesh = plsc.VectorSubcoreMesh(core_axis_name="c", subcore_axis_name="s")
    f = pl.kernel(
        _edge_agg_body,
        out_type=jax.ShapeDtypeStruct((NC, N, D), jnp.float32),
        mesh=mesh,
        scratch_types=[
            pltpu.VMEM((GCH, CHUNK), jnp.int32),
            pltpu.VMEM((GCH, CHUNK), jnp.int32),
            pltpu.VMEM((CHUNK, D), jnp.float32),
            pltpu.VMEM((CHUNK, D), jnp.float32),
            pltpu.VMEM_SHARED((N, D), jnp.float32),
            pltpu.SemaphoreType.DMA,
            pltpu.SemaphoreType.DMA,
            pltpu.SemaphoreType.DMA,
            pltpu.SemaphoreType.DMA,
        ],
    )
    return f(table, zeros, srcf, dstf, srct, dstt)


def _tc1_body(acc_ref, w_ref, b_ref, o_ref):
    h = acc_ref[0] + acc_ref[1]
    o_ref[...] = jnp.maximum(
        jnp.dot(h, w_ref[...], preferred_element_type=jnp.float32)
        + b_ref[...], 0.0)


def _tc2_body(acc_ref, w_ref, b_ref, gid_ref, lab_ref,
              emb_ref, proto_ref, sim_ref):
    h = acc_ref[0] + acc_ref[1]
    h2 = jnp.dot(h, w_ref[...], preferred_element_type=jnp.float32) + b_ref[...]
    gid = gid_ref[...]
    giota = lax.broadcasted_iota(jnp.int32, (G, N), 0)
    onehot = (gid == giota).astype(jnp.float32)
    g_sum = jnp.dot(onehot, h2, preferred_element_type=jnp.float32)
    g_cnt = jnp.sum(onehot, axis=1, keepdims=True)
    emb = g_sum / jnp.maximum(g_cnt, 1.0)
    lab = lab_ref[...]
    ciota = lax.broadcasted_iota(jnp.int32, (C, G), 0)
    oh2 = (lab == ciota).astype(jnp.float32)
    p_sum = jnp.dot(oh2, emb, preferred_element_type=jnp.float32)
    p_cnt = jnp.sum(oh2, axis=1, keepdims=True)
    proto = p_sum / jnp.maximum(p_cnt, 1.0)
    qn = emb / (jnp.sqrt(jnp.sum(emb * emb, axis=1, keepdims=True)) + 1e-8)
    pn = proto / (jnp.sqrt(jnp.sum(proto * proto, axis=1, keepdims=True))
                  + 1e-8)
    emb_ref[...] = emb
    proto_ref[...] = proto
    sim_ref[...] = lax.dot_general(
        qn, pn, (((1,), (1,)), ((), ())),
        preferred_element_type=jnp.float32)


def kernel(x, edge_index, graph_ids, graph_labels, W1, b1, W2, b2):
    f32 = jnp.float32
    zeros = jnp.zeros((N, D), f32)
    src = edge_index[0].reshape(NW, EPT)
    dst = edge_index[1].reshape(NW, EPT)
    full = GROUPS * GCH * CHUNK
    srcf = src[:, :full].reshape(NW, GROUPS, GCH, CHUNK)
    dstf = dst[:, :full].reshape(NW, GROUPS, GCH, CHUNK)
    srct = src[:, full:]
    dstt = dst[:, full:]

    acc1 = _edge_agg(x, zeros, srcf, dstf, srct, dstt)
    h1 = pl.pallas_call(
        _tc1_body,
        out_shape=jax.ShapeDtypeStruct((N, D), f32),
    )(acc1, W1, b1.reshape(1, D))

    acc2 = _edge_agg(h1, zeros, srcf, dstf, srct, dstt)
    gid = graph_ids.reshape(1, N)
    lab = graph_labels.reshape(1, G)
    embedded, prototypes, similarities = pl.pallas_call(
        _tc2_body,
        out_shape=(
            jax.ShapeDtypeStruct((G, D), f32),
            jax.ShapeDtypeStruct((C, D), f32),
            jax.ShapeDtypeStruct((G, C), f32),
        ),
    )(acc2, W2, b2.reshape(1, D), gid, lab)
    return (embedded, prototypes, similarities)


# speedup vs baseline: 1.0275x; 1.0275x over previous
import jax
import jax.numpy as jnp
from jax import lax
from jax.experimental import pallas as pl
from jax.experimental.pallas import tpu as pltpu
from jax.experimental.pallas import tpu_sc as plsc

N = 10000
E = 320000
D = 128
G = 200
C = 5

NC = 2
NS = 16
NW = NC * NS
EPT = E // NW
CHUNK = 125
GCH = 40
GROUPS = 2
RPT = 632
RPT_LAST = N - (NS - 1) * RPT


def _edge_agg_body(table_hbm, zeros_hbm, src_hbm, dst_hbm, out_hbm,
                   srcv, dstv, rows0, rows1, acc,
                   gsem0, gsem1, ssem0, ssem1):
    cid = lax.axis_index("c")
    sid = lax.axis_index("s")
    wid = cid * NS + sid
    lo = sid * RPT

    pltpu.sync_copy(src_hbm.at[wid, 0], srcv)
    pltpu.sync_copy(dst_hbm.at[wid, 0], dstv)
    pltpu.async_copy(table_hbm.at[srcv.at[0]], rows0, gsem0)
    pltpu.async_copy(table_hbm.at[srcv.at[1]], rows1, gsem1)

    @pl.when((cid == 0) & (sid < NS - 1))
    def _():
        pltpu.sync_copy(table_hbm.at[pl.ds(lo, RPT)], acc.at[pl.ds(lo, RPT)])

    @pl.when((cid == 0) & (sid == NS - 1))
    def _():
        pltpu.sync_copy(table_hbm.at[pl.ds((NS - 1) * RPT, RPT_LAST)],
                        acc.at[pl.ds((NS - 1) * RPT, RPT_LAST)])

    @pl.when((cid != 0) & (sid < NS - 1))
    def _():
        pltpu.sync_copy(zeros_hbm.at[pl.ds(lo, RPT)], acc.at[pl.ds(lo, RPT)])

    @pl.when((cid != 0) & (sid == NS - 1))
    def _():
        pltpu.sync_copy(zeros_hbm.at[pl.ds((NS - 1) * RPT, RPT_LAST)],
                        acc.at[pl.ds((NS - 1) * RPT, RPT_LAST)])

    plsc.subcore_barrier()

    for g in range(GROUPS):
        if g > 0:
            pltpu.sync_copy(src_hbm.at[wid, g], srcv)
            pltpu.sync_copy(dst_hbm.at[wid, g], dstv)
            pltpu.async_copy(table_hbm.at[srcv.at[0]], rows0, gsem0)
            pltpu.async_copy(table_hbm.at[srcv.at[1]], rows1, gsem1)

        @pl.loop(0, GCH, step=2)
        def _(j):
            pltpu.make_async_copy(
                table_hbm.at[srcv.at[j]], rows0, gsem0).wait()
            pltpu.async_copy(rows0, acc.at[dstv.at[j]], ssem0, add=True)
            pltpu.make_async_copy(
                table_hbm.at[srcv.at[j + 1]], rows1, gsem1).wait()
            pltpu.async_copy(rows1, acc.at[dstv.at[j + 1]], ssem1, add=True)
            pltpu.make_async_copy(rows0, acc.at[dstv.at[j]], ssem0).wait()

            @pl.when(j + 2 < GCH)
            def _():
                pltpu.async_copy(table_hbm.at[srcv.at[j + 2]], rows0, gsem0)
            pltpu.make_async_copy(rows1, acc.at[dstv.at[j + 1]], ssem1).wait()

            @pl.when(j + 3 < GCH)
            def _():
                pltpu.async_copy(table_hbm.at[srcv.at[j + 3]], rows1, gsem1)

    plsc.subcore_barrier()

    @pl.when(sid < NS - 1)
    def _():
        pltpu.sync_copy(acc.at[pl.ds(lo, RPT)],
                        out_hbm.at[cid, pl.ds(lo, RPT)])

    @pl.when(sid == NS - 1)
    def _():
        pltpu.sync_copy(acc.at[pl.ds((NS - 1) * RPT, RPT_LAST)],
                        out_hbm.at[cid, pl.ds((NS - 1) * RPT, RPT_LAST)])


def _edge_agg(table, zeros, srcf, dstf):
    mesh = plsc.VectorSubcoreMesh(core_axis_name="c", subcore_axis_name="s")
    f = pl.kernel(
        _edge_agg_body,
        out_type=jax.ShapeDtypeStruct((NC, N, D), jnp.float32),
        mesh=mesh,
        scratch_types=[
            pltpu.VMEM((GCH, CHUNK), jnp.int32),
            pltpu.VMEM((GCH, CHUNK), jnp.int32),
            pltpu.VMEM((CHUNK, D), jnp.float32),
            pltpu.VMEM((CHUNK, D), jnp.float32),
            pltpu.VMEM_SHARED((N, D), jnp.float32),
            pltpu.SemaphoreType.DMA,
            pltpu.SemaphoreType.DMA,
            pltpu.SemaphoreType.DMA,
            pltpu.SemaphoreType.DMA,
        ],
    )
    return f(table, zeros, srcf, dstf)


def _tc1_body(acc_ref, w_ref, b_ref, o_ref):
    h = acc_ref[0] + acc_ref[1]
    o_ref[...] = jnp.maximum(
        jnp.dot(h, w_ref[...], preferred_element_type=jnp.float32)
        + b_ref[...], 0.0)


def _tc2_body(acc_ref, w_ref, b_ref, gid_ref, lab_ref,
              emb_ref, proto_ref, sim_ref):
    h = acc_ref[0] + acc_ref[1]
    h2 = jnp.dot(h, w_ref[...], preferred_element_type=jnp.float32) + b_ref[...]
    gid = gid_ref[...]
    giota = lax.broadcasted_iota(jnp.int32, (G, N), 0)
    onehot = (gid == giota).astype(jnp.float32)
    g_sum = jnp.dot(onehot, h2, preferred_element_type=jnp.float32)
    g_cnt = jnp.sum(onehot, axis=1, keepdims=True)
    emb = g_sum / jnp.maximum(g_cnt, 1.0)
    lab = lab_ref[...]
    ciota = lax.broadcasted_iota(jnp.int32, (C, G), 0)
    oh2 = (lab == ciota).astype(jnp.float32)
    p_sum = jnp.dot(oh2, emb, preferred_element_type=jnp.float32)
    p_cnt = jnp.sum(oh2, axis=1, keepdims=True)
    proto = p_sum / jnp.maximum(p_cnt, 1.0)
    qn = emb / (jnp.sqrt(jnp.sum(emb * emb, axis=1, keepdims=True)) + 1e-8)
    pn = proto / (jnp.sqrt(jnp.sum(proto * proto, axis=1, keepdims=True))
                  + 1e-8)
    emb_ref[...] = emb
    proto_ref[...] = proto
    sim_ref[...] = lax.dot_general(
        qn, pn, (((1,), (1,)), ((), ())),
        preferred_element_type=jnp.float32)


def kernel(x, edge_index, graph_ids, graph_labels, W1, b1, W2, b2):
    f32 = jnp.float32
    zeros = jnp.zeros((N, D), f32)
    srcf = edge_index[0].reshape(NW, GROUPS, GCH, CHUNK)
    dstf = edge_index[1].reshape(NW, GROUPS, GCH, CHUNK)

    acc1 = _edge_agg(x, zeros, srcf, dstf)
    h1 = pl.pallas_call(
        _tc1_body,
        out_shape=jax.ShapeDtypeStruct((N, D), f32),
    )(acc1, W1, b1.reshape(1, D))

    acc2 = _edge_agg(h1, zeros, srcf, dstf)
    gid = graph_ids.reshape(1, N)
    lab = graph_labels.reshape(1, G)
    embedded, prototypes, similarities = pl.pallas_call(
        _tc2_body,
        out_shape=(
            jax.ShapeDtypeStruct((G, D), f32),
            jax.ShapeDtypeStruct((C, D), f32),
            jax.ShapeDtypeStruct((G, C), f32),
        ),
    )(acc2, W2, b2.reshape(1, D), gid, lab)
    return (embedded, prototypes, similarities)


# speedup vs baseline: 1.0411x; 1.0132x over previous
import jax
import jax.numpy as jnp
from jax import lax
from jax.experimental import pallas as pl
from jax.experimental.pallas import tpu as pltpu
from jax.experimental.pallas import tpu_sc as plsc

N = 10000
E = 320000
D = 128
G = 200
C = 5

NC = 2
NS = 16
NW = NC * NS
EPT = E // NW
CHUNK = 125
GCH = 40
GROUPS = 2
RPT = 632
RPT_LAST = N - (NS - 1) * RPT


HA = 64
HB = CHUNK - HA


def _edge_agg_body(table_hbm, zeros_hbm, src_hbm, dst_hbm, out_hbm,
                   srcv, dstv, rows0, rows1, acc,
                   gsem0a, gsem0b, gsem1a, gsem1b, ssem0, ssem1):
    cid = lax.axis_index("c")
    sid = lax.axis_index("s")
    wid = cid * NS + sid
    lo = sid * RPT

    def gather(j, rows, sema, semb):
        pltpu.async_copy(table_hbm.at[srcv.at[j, pl.ds(0, HA)]],
                         rows.at[pl.ds(0, HA)], sema)
        pltpu.async_copy(table_hbm.at[srcv.at[j, pl.ds(HA, HB)]],
                         rows.at[pl.ds(HA, HB)], semb)

    def gather_wait(j, rows, sema, semb):
        pltpu.make_async_copy(table_hbm.at[srcv.at[j, pl.ds(0, HA)]],
                              rows.at[pl.ds(0, HA)], sema).wait()
        pltpu.make_async_copy(table_hbm.at[srcv.at[j, pl.ds(HA, HB)]],
                              rows.at[pl.ds(HA, HB)], semb).wait()

    pltpu.sync_copy(src_hbm.at[wid, 0], srcv)
    pltpu.sync_copy(dst_hbm.at[wid, 0], dstv)
    gather(0, rows0, gsem0a, gsem0b)
    gather(1, rows1, gsem1a, gsem1b)

    @pl.when((cid == 0) & (sid < NS - 1))
    def _():
        pltpu.sync_copy(table_hbm.at[pl.ds(lo, RPT)], acc.at[pl.ds(lo, RPT)])

    @pl.when((cid == 0) & (sid == NS - 1))
    def _():
        pltpu.sync_copy(table_hbm.at[pl.ds((NS - 1) * RPT, RPT_LAST)],
                        acc.at[pl.ds((NS - 1) * RPT, RPT_LAST)])

    @pl.when((cid != 0) & (sid < NS - 1))
    def _():
        pltpu.sync_copy(zeros_hbm.at[pl.ds(lo, RPT)], acc.at[pl.ds(lo, RPT)])

    @pl.when((cid != 0) & (sid == NS - 1))
    def _():
        pltpu.sync_copy(zeros_hbm.at[pl.ds((NS - 1) * RPT, RPT_LAST)],
                        acc.at[pl.ds((NS - 1) * RPT, RPT_LAST)])

    plsc.subcore_barrier()

    for g in range(GROUPS):
        if g > 0:
            pltpu.sync_copy(src_hbm.at[wid, g], srcv)
            pltpu.sync_copy(dst_hbm.at[wid, g], dstv)
            gather(0, rows0, gsem0a, gsem0b)
            gather(1, rows1, gsem1a, gsem1b)

        @pl.loop(0, GCH, step=2)
        def _(j):
            gather_wait(j, rows0, gsem0a, gsem0b)
            pltpu.async_copy(rows0, acc.at[dstv.at[j]], ssem0, add=True)
            gather_wait(j + 1, rows1, gsem1a, gsem1b)
            pltpu.async_copy(rows1, acc.at[dstv.at[j + 1]], ssem1, add=True)
            pltpu.make_async_copy(rows0, acc.at[dstv.at[j]], ssem0).wait()

            @pl.when(j + 2 < GCH)
            def _():
                gather(j + 2, rows0, gsem0a, gsem0b)
            pltpu.make_async_copy(rows1, acc.at[dstv.at[j + 1]], ssem1).wait()

            @pl.when(j + 3 < GCH)
            def _():
                gather(j + 3, rows1, gsem1a, gsem1b)

    plsc.subcore_barrier()

    @pl.when(sid < NS - 1)
    def _():
        pltpu.sync_copy(acc.at[pl.ds(lo, RPT)],
                        out_hbm.at[cid, pl.ds(lo, RPT)])

    @pl.when(sid == NS - 1)
    def _():
        pltpu.sync_copy(acc.at[pl.ds((NS - 1) * RPT, RPT_LAST)],
                        out_hbm.at[cid, pl.ds((NS - 1) * RPT, RPT_LAST)])


def _edge_agg(table, zeros, srcf, dstf):
    mesh = plsc.VectorSubcoreMesh(core_axis_name="c", subcore_axis_name="s")
    f = pl.kernel(
        _edge_agg_body,
        out_type=jax.ShapeDtypeStruct((NC, N, D), jnp.float32),
        mesh=mesh,
        scratch_types=[
            pltpu.VMEM((GCH, CHUNK), jnp.int32),
            pltpu.VMEM((GCH, CHUNK), jnp.int32),
            pltpu.VMEM((CHUNK, D), jnp.float32),
            pltpu.VMEM((CHUNK, D), jnp.float32),
            pltpu.VMEM_SHARED((N, D), jnp.float32),
            pltpu.SemaphoreType.DMA,
            pltpu.SemaphoreType.DMA,
            pltpu.SemaphoreType.DMA,
            pltpu.SemaphoreType.DMA,
            pltpu.SemaphoreType.DMA,
            pltpu.SemaphoreType.DMA,
        ],
    )
    return f(table, zeros, srcf, dstf)


def _tc1_body(acc_ref, w_ref, b_ref, o_ref):
    h = acc_ref[0] + acc_ref[1]
    o_ref[...] = jnp.maximum(
        jnp.dot(h, w_ref[...], preferred_element_type=jnp.float32)
        + b_ref[...], 0.0)


def _tc2_body(acc_ref, w_ref, b_ref, gid_ref, lab_ref,
              emb_ref, proto_ref, sim_ref):
    h = acc_ref[0] + acc_ref[1]
    h2 = jnp.dot(h, w_ref[...], preferred_element_type=jnp.float32) + b_ref[...]
    gid = gid_ref[...]
    giota = lax.broadcasted_iota(jnp.int32, (G, N), 0)
    onehot = (gid == giota).astype(jnp.float32)
    g_sum = jnp.dot(onehot, h2, preferred_element_type=jnp.float32)
    g_cnt = jnp.sum(onehot, axis=1, keepdims=True)
    emb = g_sum / jnp.maximum(g_cnt, 1.0)
    lab = lab_ref[...]
    ciota = lax.broadcasted_iota(jnp.int32, (C, G), 0)
    oh2 = (lab == ciota).astype(jnp.float32)
    p_sum = jnp.dot(oh2, emb, preferred_element_type=jnp.float32)
    p_cnt = jnp.sum(oh2, axis=1, keepdims=True)
    proto = p_sum / jnp.maximum(p_cnt, 1.0)
    qn = emb / (jnp.sqrt(jnp.sum(emb * emb, axis=1, keepdims=True)) + 1e-8)
    pn = proto / (jnp.sqrt(jnp.sum(proto * proto, axis=1, keepdims=True))
                  + 1e-8)
    emb_ref[...] = emb
    proto_ref[...] = proto
    sim_ref[...] = lax.dot_general(
        qn, pn, (((1,), (1,)), ((), ())),
        preferred_element_type=jnp.float32)


def kernel(x, edge_index, graph_ids, graph_labels, W1, b1, W2, b2):
    f32 = jnp.float32
    zeros = jnp.zeros((N, D), f32)
    srcf = edge_index[0].reshape(NW, GROUPS, GCH, CHUNK)
    dstf = edge_index[1].reshape(NW, GROUPS, GCH, CHUNK)

    acc1 = _edge_agg(x, zeros, srcf, dstf)
    h1 = pl.pallas_call(
        _tc1_body,
        out_shape=jax.ShapeDtypeStruct((N, D), f32),
    )(acc1, W1, b1.reshape(1, D))

    acc2 = _edge_agg(h1, zeros, srcf, dstf)
    gid = graph_ids.reshape(1, N)
    lab = graph_labels.reshape(1, G)
    embedded, prototypes, similarities = pl.pallas_call(
        _tc2_body,
        out_shape=(
            jax.ShapeDtypeStruct((G, D), f32),
            jax.ShapeDtypeStruct((C, D), f32),
            jax.ShapeDtypeStruct((G, C), f32),
        ),
    )(acc2, W2, b2.reshape(1, D), gid, lab)
    return (embedded, prototypes, similarities)


# speedup vs baseline: 1.2213x; 1.1732x over previous
import jax
import jax.numpy as jnp
from jax import lax
from jax.experimental import pallas as pl
from jax.experimental.pallas import tpu as pltpu
from jax.experimental.pallas import tpu_sc as plsc

N = 10000
E = 320000
D = 128
G = 200
C = 5

NC = 2
NS = 16
NW = NC * NS
EPT = E // NW
CHUNK = 125
GCH = 40
GROUPS = 2
RPT = 632
RPT_LAST = N - (NS - 1) * RPT


HA = 64
HB = CHUNK - HA


def _edge_agg_body(table_hbm, zeros_hbm, src_hbm, dst_hbm, out_hbm,
                   srcv, dstv, rows0, rows1, acc,
                   gsem0a, gsem0b, gsem1a, gsem1b,
                   ssem0a, ssem0b, ssem1a, ssem1b):
    cid = lax.axis_index("c")
    sid = lax.axis_index("s")
    wid = cid * NS + sid
    lo = sid * RPT

    def gather(j, rows, sema, semb):
        pltpu.async_copy(table_hbm.at[srcv.at[j, pl.ds(0, HA)]],
                         rows.at[pl.ds(0, HA)], sema)
        pltpu.async_copy(table_hbm.at[srcv.at[j, pl.ds(HA, HB)]],
                         rows.at[pl.ds(HA, HB)], semb)

    def gather_wait(j, rows, sema, semb):
        pltpu.make_async_copy(table_hbm.at[srcv.at[j, pl.ds(0, HA)]],
                              rows.at[pl.ds(0, HA)], sema).wait()
        pltpu.make_async_copy(table_hbm.at[srcv.at[j, pl.ds(HA, HB)]],
                              rows.at[pl.ds(HA, HB)], semb).wait()

    def scatter(j, rows, sema, semb):
        pltpu.async_copy(rows.at[pl.ds(0, HA)],
                         acc.at[dstv.at[j, pl.ds(0, HA)]], sema, add=True)
        pltpu.async_copy(rows.at[pl.ds(HA, HB)],
                         acc.at[dstv.at[j, pl.ds(HA, HB)]], semb, add=True)

    def scatter_wait(j, rows, sema, semb):
        pltpu.make_async_copy(rows.at[pl.ds(0, HA)],
                              acc.at[dstv.at[j, pl.ds(0, HA)]], sema).wait()
        pltpu.make_async_copy(rows.at[pl.ds(HA, HB)],
                              acc.at[dstv.at[j, pl.ds(HA, HB)]], semb).wait()

    pltpu.sync_copy(src_hbm.at[wid, 0], srcv)
    pltpu.sync_copy(dst_hbm.at[wid, 0], dstv)
    gather(0, rows0, gsem0a, gsem0b)
    gather(1, rows1, gsem1a, gsem1b)

    @pl.when((cid == 0) & (sid < NS - 1))
    def _():
        pltpu.sync_copy(table_hbm.at[pl.ds(lo, RPT)], acc.at[pl.ds(lo, RPT)])

    @pl.when((cid == 0) & (sid == NS - 1))
    def _():
        pltpu.sync_copy(table_hbm.at[pl.ds((NS - 1) * RPT, RPT_LAST)],
                        acc.at[pl.ds((NS - 1) * RPT, RPT_LAST)])

    @pl.when((cid != 0) & (sid < NS - 1))
    def _():
        pltpu.sync_copy(zeros_hbm.at[pl.ds(lo, RPT)], acc.at[pl.ds(lo, RPT)])

    @pl.when((cid != 0) & (sid == NS - 1))
    def _():
        pltpu.sync_copy(zeros_hbm.at[pl.ds((NS - 1) * RPT, RPT_LAST)],
                        acc.at[pl.ds((NS - 1) * RPT, RPT_LAST)])

    plsc.subcore_barrier()

    for g in range(GROUPS):
        if g > 0:
            pltpu.sync_copy(src_hbm.at[wid, g], srcv)
            pltpu.sync_copy(dst_hbm.at[wid, g], dstv)
            gather(0, rows0, gsem0a, gsem0b)
            gather(1, rows1, gsem1a, gsem1b)

        @pl.loop(0, GCH, step=2)
        def _(j):
            gather_wait(j, rows0, gsem0a, gsem0b)
            scatter(j, rows0, ssem0a, ssem0b)
            gather_wait(j + 1, rows1, gsem1a, gsem1b)
            scatter(j + 1, rows1, ssem1a, ssem1b)
            scatter_wait(j, rows0, ssem0a, ssem0b)

            @pl.when(j + 2 < GCH)
            def _():
                gather(j + 2, rows0, gsem0a, gsem0b)
            scatter_wait(j + 1, rows1, ssem1a, ssem1b)

            @pl.when(j + 3 < GCH)
            def _():
                gather(j + 3, rows1, gsem1a, gsem1b)

    plsc.subcore_barrier()

    @pl.when(sid < NS - 1)
    def _():
        pltpu.sync_copy(acc.at[pl.ds(lo, RPT)],
                        out_hbm.at[cid, pl.ds(lo, RPT)])

    @pl.when(sid == NS - 1)
    def _():
        pltpu.sync_copy(acc.at[pl.ds((NS - 1) * RPT, RPT_LAST)],
                        out_hbm.at[cid, pl.ds((NS - 1) * RPT, RPT_LAST)])


def _edge_agg(table, zeros, srcf, dstf):
    mesh = plsc.VectorSubcoreMesh(core_axis_name="c", subcore_axis_name="s")
    f = pl.kernel(
        _edge_agg_body,
        out_type=jax.ShapeDtypeStruct((NC, N, D), jnp.float32),
        mesh=mesh,
        scratch_types=[
            pltpu.VMEM((GCH, CHUNK), jnp.int32),
            pltpu.VMEM((GCH, CHUNK), jnp.int32),
            pltpu.VMEM((CHUNK, D), jnp.float32),
            pltpu.VMEM((CHUNK, D), jnp.float32),
            pltpu.VMEM_SHARED((N, D), jnp.float32),
            pltpu.SemaphoreType.DMA,
            pltpu.SemaphoreType.DMA,
            pltpu.SemaphoreType.DMA,
            pltpu.SemaphoreType.DMA,
            pltpu.SemaphoreType.DMA,
            pltpu.SemaphoreType.DMA,
            pltpu.SemaphoreType.DMA,
            pltpu.SemaphoreType.DMA,
        ],
    )
    return f(table, zeros, srcf, dstf)


def _tc1_body(acc_ref, w_ref, b_ref, o_ref):
    h = acc_ref[0] + acc_ref[1]
    o_ref[...] = jnp.maximum(
        jnp.dot(h, w_ref[...], preferred_element_type=jnp.float32)
        + b_ref[...], 0.0)


def _tc2_body(acc_ref, w_ref, b_ref, gid_ref, lab_ref,
              emb_ref, proto_ref, sim_ref):
    h = acc_ref[0] + acc_ref[1]
    h2 = jnp.dot(h, w_ref[...], preferred_element_type=jnp.float32) + b_ref[...]
    gid = gid_ref[...]
    giota = lax.broadcasted_iota(jnp.int32, (G, N), 0)
    onehot = (gid == giota).astype(jnp.float32)
    g_sum = jnp.dot(onehot, h2, preferred_element_type=jnp.float32)
    g_cnt = jnp.sum(onehot, axis=1, keepdims=True)
    emb = g_sum / jnp.maximum(g_cnt, 1.0)
    lab = lab_ref[...]
    ciota = lax.broadcasted_iota(jnp.int32, (C, G), 0)
    oh2 = (lab == ciota).astype(jnp.float32)
    p_sum = jnp.dot(oh2, emb, preferred_element_type=jnp.float32)
    p_cnt = jnp.sum(oh2, axis=1, keepdims=True)
    proto = p_sum / jnp.maximum(p_cnt, 1.0)
    qn = emb / (jnp.sqrt(jnp.sum(emb * emb, axis=1, keepdims=True)) + 1e-8)
    pn = proto / (jnp.sqrt(jnp.sum(proto * proto, axis=1, keepdims=True))
                  + 1e-8)
    emb_ref[...] = emb
    proto_ref[...] = proto
    sim_ref[...] = lax.dot_general(
        qn, pn, (((1,), (1,)), ((), ())),
        preferred_element_type=jnp.float32)


def kernel(x, edge_index, graph_ids, graph_labels, W1, b1, W2, b2):
    f32 = jnp.float32
    zeros = jnp.zeros((N, D), f32)
    srcf = edge_index[0].reshape(NW, GROUPS, GCH, CHUNK)
    dstf = edge_index[1].reshape(NW, GROUPS, GCH, CHUNK)

    acc1 = _edge_agg(x, zeros, srcf, dstf)
    h1 = pl.pallas_call(
        _tc1_body,
        out_shape=jax.ShapeDtypeStruct((N, D), f32),
    )(acc1, W1, b1.reshape(1, D))

    acc2 = _edge_agg(h1, zeros, srcf, dstf)
    gid = graph_ids.reshape(1, N)
    lab = graph_labels.reshape(1, G)
    embedded, prototypes, similarities = pl.pallas_call(
        _tc2_body,
        out_shape=(
            jax.ShapeDtypeStruct((G, D), f32),
            jax.ShapeDtypeStruct((C, D), f32),
            jax.ShapeDtypeStruct((G, C), f32),
        ),
    )(acc2, W2, b2.reshape(1, D), gid, lab)
    return (embedded, prototypes, similarities)
